# baseline (speedup 1.0000x reference)
import jax
import jax.numpy as jnp
from jax.experimental import pallas as pl
from jax.experimental.pallas import tpu as pltpu

_NUM_CHANNELS = 5


def _body(x_ref, e4_ref, u_ref, xo_ref, eo_ref, uo_ref, sems):
    de = eo_ref.shape[0]

    copies = []
    for ch in range(de):
        cp = pltpu.make_async_copy(
            e4_ref.at[ch // 8, :, :, ch % 8, :],
            eo_ref.at[ch],
            sems.at[ch],
        )
        cp.start()
        copies.append(cp)

    scale = jnp.float32(1.0 / _NUM_CHANNELS)

    def channel_mean(v):
        acc = v
        for _ in range(_NUM_CHANNELS - 1):
            acc = acc + v
        return acc * scale

    xo_ref[...] = channel_mean(x_ref[...])
    uo_ref[...] = channel_mean(u_ref[...])

    for cp in copies:
        cp.wait()


def kernel(x, edge_index, edge_attr, u, batch):
    n, d = x.shape
    e, de = edge_attr.shape

    et = edge_attr.T
    e4 = et.reshape(de // 8, 8, e // 128, 1, 128).transpose(0, 2, 3, 1, 4)

    x_out, e3, u_out = pl.pallas_call(
        _body,
        in_specs=[
            pl.BlockSpec(memory_space=pltpu.VMEM),
            pl.BlockSpec(memory_space=pl.ANY),
            pl.BlockSpec(memory_space=pltpu.VMEM),
        ],
        out_specs=[
            pl.BlockSpec(memory_space=pltpu.VMEM),
            pl.BlockSpec(memory_space=pl.ANY),
            pl.BlockSpec(memory_space=pltpu.VMEM),
        ],
        out_shape=[
            jax.ShapeDtypeStruct((n, d), x.dtype),
            jax.ShapeDtypeStruct((de, e // 128, 1, 128), edge_attr.dtype),
            jax.ShapeDtypeStruct((1, d), u.dtype),
        ],
        scratch_shapes=[pltpu.SemaphoreType.DMA((de,))],
    )(x, e4, u)

    e_out = e3.reshape(de, e, 1).transpose(1, 0, 2)

    return (x_out[:, :, None], e_out, u_out[:, :, None])


# speedup vs baseline: 20.4989x; 20.4989x over previous
import jax
import jax.numpy as jnp
from jax.experimental import pallas as pl
from jax.experimental.pallas import tpu as pltpu

_NUM_CHANNELS = 5
_GRID = 10


def _scale():
    return jnp.float32(1.0 / _NUM_CHANNELS)


def _channel_mean(v):
    return (v * jnp.float32(_NUM_CHANNELS)) * _scale()


def _body(x_ref, e_ref, u_ref, xo_ref, eo_ref, uo_ref):
    ev = e_ref[...]
    cb = ev.shape[1]
    evt = ev.transpose(0, 3, 1, 2, 4).reshape(16, cb, 1, 128)
    eo_ref[...] = _channel_mean(evt)

    xo_ref[...] = _channel_mean(x_ref[...])
    uo_ref[...] = _channel_mean(u_ref[...])


def kernel(x, edge_index, edge_attr, u, batch):
    n, d = x.shape
    e, de = edge_attr.shape

    et = edge_attr.T
    e4 = et.reshape(de // 8, 8, e // 128, 1, 128).transpose(0, 2, 3, 1, 4)

    xb = n // _GRID
    cb = (e // 128) // _GRID

    x_out, e3, u_out = pl.pallas_call(
        _body,
        grid=(_GRID,),
        in_specs=[
            pl.BlockSpec((xb, d), lambda i: (i, 0)),
            pl.BlockSpec((de // 8, cb, 1, 8, 128), lambda i: (0, i, 0, 0, 0)),
            pl.BlockSpec((1, d), lambda i: (0, 0)),
        ],
        out_specs=[
            pl.BlockSpec((xb, d), lambda i: (i, 0)),
            pl.BlockSpec((de, cb, 1, 128), lambda i: (0, i, 0, 0)),
            pl.BlockSpec((1, d), lambda i: (0, 0)),
        ],
        out_shape=[
            jax.ShapeDtypeStruct((n, d), x.dtype),
            jax.ShapeDtypeStruct((de, e // 128, 1, 128), edge_attr.dtype),
            jax.ShapeDtypeStruct((1, d), u.dtype),
        ],
        compiler_params=pltpu.CompilerParams(
            dimension_semantics=("arbitrary",),
        ),
    )(x, e4, u)

    e_out = e3.reshape(de, e, 1).transpose(1, 0, 2)
    return (x_out[:, :, None], e_out, u_out[:, :, None])


# speedup vs baseline: 23.2195x; 1.1327x over previous
import jax
import jax.numpy as jnp
from jax.experimental import pallas as pl
from jax.experimental.pallas import tpu as pltpu

_NUM_CHANNELS = 5
_GRID = 10


def _scale():
    return jnp.float32(1.0 / _NUM_CHANNELS)


def _channel_mean(v):
    return (v * jnp.float32(_NUM_CHANNELS)) * _scale()


def _body(x_ref, e_ref, u_ref, xo_ref, eo_ref, uo_ref):
    ev = e_ref[...]
    cb = ev.shape[1]
    evt = ev.transpose(0, 3, 1, 2, 4).reshape(16, cb, 1, 128)
    eo_ref[...] = evt

    xo_ref[...] = _channel_mean(x_ref[...])
    uo_ref[...] = _channel_mean(u_ref[...])


def kernel(x, edge_index, edge_attr, u, batch):
    n, d = x.shape
    e, de = edge_attr.shape

    et = edge_attr.T
    e4 = et.reshape(de // 8, 8, e // 128, 1, 128).transpose(0, 2, 3, 1, 4)

    xb = n // _GRID
    cb = (e // 128) // _GRID

    x_out, e3, u_out = pl.pallas_call(
        _body,
        grid=(_GRID,),
        in_specs=[
            pl.BlockSpec((xb, d), lambda i: (i, 0)),
            pl.BlockSpec((de // 8, cb, 1, 8, 128), lambda i: (0, i, 0, 0, 0)),
            pl.BlockSpec((1, d), lambda i: (0, 0)),
        ],
        out_specs=[
            pl.BlockSpec((xb, d), lambda i: (i, 0)),
            pl.BlockSpec((de, cb, 1, 128), lambda i: (0, i, 0, 0)),
            pl.BlockSpec((1, d), lambda i: (0, 0)),
        ],
        out_shape=[
            jax.ShapeDtypeStruct((n, d), x.dtype),
            jax.ShapeDtypeStruct((de, e // 128, 1, 128), edge_attr.dtype),
            jax.ShapeDtypeStruct((1, d), u.dtype),
        ],
        compiler_params=pltpu.CompilerParams(
            dimension_semantics=("arbitrary",),
        ),
    )(x, e4, u)

    e_out = e3.reshape(de, e, 1).transpose(1, 0, 2)
    return (x_out[:, :, None], e_out, u_out[:, :, None])


# speedup vs baseline: 23.3693x; 1.0065x over previous
import jax
import jax.numpy as jnp
from jax.experimental import pallas as pl
from jax.experimental.pallas import tpu as pltpu

_NUM_CHANNELS = 5
_GRID = 10


def _scale():
    return jnp.float32(1.0 / _NUM_CHANNELS)


def _channel_mean(v):
    return (v * jnp.float32(_NUM_CHANNELS)) * _scale()


def _body(x_ref, e_ref, u_ref, xo_ref, eo_ref, uo_ref):
    ev = e_ref[...]
    cb = ev.shape[1]
    evt = ev.transpose(0, 3, 1, 2, 4).reshape(16, cb, 1, 128)
    eo_ref[...] = evt

    xo_ref[...] = _channel_mean(x_ref[...])
    uo_ref[...] = _channel_mean(u_ref[...])


def kernel(x, edge_index, edge_attr, u, batch):
    n, d = x.shape
    e, de = edge_attr.shape

    et = edge_attr.T
    e4 = et.reshape(de // 8, 8, e // 128, 1, 128).transpose(0, 2, 3, 1, 4)

    xb = n // _GRID
    cb = (e // 128) // _GRID

    x_out, e3, u_out = pl.pallas_call(
        _body,
        grid=(_GRID,),
        in_specs=[
            pl.BlockSpec((xb, d), lambda i: (i, 0)),
            pl.BlockSpec((de // 8, cb, 1, 8, 128), lambda i: (0, i, 0, 0, 0)),
            pl.BlockSpec((1, d), lambda i: (0, 0)),
        ],
        out_specs=[
            pl.BlockSpec((xb, d), lambda i: (i, 0)),
            pl.BlockSpec((de, cb, 1, 128), lambda i: (0, i, 0, 0)),
            pl.BlockSpec((1, d), lambda i: (0, 0)),
        ],
        out_shape=[
            jax.ShapeDtypeStruct((n, d), x.dtype),
            jax.ShapeDtypeStruct((de, e // 128, 1, 128), edge_attr.dtype),
            jax.ShapeDtypeStruct((1, d), u.dtype),
        ],
        compiler_params=pltpu.CompilerParams(
            dimension_semantics=("parallel",),
        ),
    )(x, e4, u)

    e_out = e3.reshape(de, e, 1).transpose(1, 0, 2)
    return (x_out[:, :, None], e_out, u_out[:, :, None])


# speedup vs baseline: 24.0332x; 1.0284x over previous
import jax
import jax.numpy as jnp
from jax.experimental import pallas as pl
from jax.experimental.pallas import tpu as pltpu

_NUM_CHANNELS = 5
_GRID = 5


def _scale():
    return jnp.float32(1.0 / _NUM_CHANNELS)


def _channel_mean(v):
    return (v * jnp.float32(_NUM_CHANNELS)) * _scale()


def _body(x_ref, e_ref, u_ref, xo_ref, eo_ref, uo_ref):
    ev = e_ref[...]
    cb = ev.shape[1]
    evt = ev.transpose(0, 3, 1, 2, 4).reshape(16, cb, 1, 128)
    eo_ref[...] = evt

    xo_ref[...] = _channel_mean(x_ref[...])
    uo_ref[...] = _channel_mean(u_ref[...])


def kernel(x, edge_index, edge_attr, u, batch):
    n, d = x.shape
    e, de = edge_attr.shape

    et = edge_attr.T
    e4 = et.reshape(de // 8, 8, e // 128, 1, 128).transpose(0, 2, 3, 1, 4)

    xb = n // _GRID
    cb = (e // 128) // _GRID

    x_out, e3, u_out = pl.pallas_call(
        _body,
        grid=(_GRID,),
        in_specs=[
            pl.BlockSpec((xb, d), lambda i: (i, 0)),
            pl.BlockSpec((de // 8, cb, 1, 8, 128), lambda i: (0, i, 0, 0, 0)),
            pl.BlockSpec((1, d), lambda i: (0, 0)),
        ],
        out_specs=[
            pl.BlockSpec((xb, d), lambda i: (i, 0)),
            pl.BlockSpec((de, cb, 1, 128), lambda i: (0, i, 0, 0)),
            pl.BlockSpec((1, d), lambda i: (0, 0)),
        ],
        out_shape=[
            jax.ShapeDtypeStruct((n, d), x.dtype),
            jax.ShapeDtypeStruct((de, e // 128, 1, 128), edge_attr.dtype),
            jax.ShapeDtypeStruct((1, d), u.dtype),
        ],
        compiler_params=pltpu.CompilerParams(
            dimension_semantics=("arbitrary",),
        ),
    )(x, e4, u)

    e_out = e3.reshape(de, e, 1).transpose(1, 0, 2)
    return (x_out[:, :, None], e_out, u_out[:, :, None])
